# baseline (speedup 1.0000x reference)
import functools

import jax
import jax.numpy as jnp
from jax import lax
from jax.experimental import pallas as pl
from jax.experimental.pallas import tpu as pltpu
from jax.experimental.pallas import tpu_sc as plsc

K = 2
DIM = 32
TEMP = 0.1
NUM_CATES = 100


def _sc_gather_fn(B):
    info = plsc.get_sparse_core_info()
    NC, NS = info.num_cores, info.num_subcores
    NW = NC * NS
    b_per_w = B // NW
    mesh = plsc.VectorSubcoreMesh(core_axis_name="c", subcore_axis_name="s")

    @functools.partial(
        pl.kernel,
        mesh=mesh,
        out_type=jax.ShapeDtypeStruct((B, 128), jnp.float32),
        scratch_types=[
            pltpu.VMEM((b_per_w,), jnp.int32),
            pltpu.VMEM((b_per_w,), jnp.int32),
            pltpu.VMEM((b_per_w, 128), jnp.float32),
            pltpu.SemaphoreType.DMA,
        ],
    )
    def gather(table_hbm, idx_hbm, out_hbm, ids_v, idx2_v, rows_v, sem):
        wid = lax.axis_index("s") * NC + lax.axis_index("c")
        base = wid * b_per_w
        pltpu.sync_copy(idx_hbm.at[pl.ds(base, b_per_w)], ids_v)
        for j in range(b_per_w // 16):
            sl = pl.ds(j * 16, 16)
            idx2_v[sl] = lax.shift_right_logical(ids_v[sl], 1)
        pltpu.async_copy(table_hbm.at[idx2_v], rows_v, sem).wait()
        pltpu.sync_copy(rows_v, out_hbm.at[pl.ds(base, b_per_w)])

    return gather


def _tc_body(u_ref, uid_ref, cid_ref, item_ref, cate_ref, gv_ref, out_ref):
    u128 = u_ref[...]
    uid = uid_ref[...]
    even = (uid & 1) == 0
    u = jnp.where(even, u128[:, 0:2 * DIM], u128[:, 2 * DIM:4 * DIM])
    cid = cid_ref[...]
    item = item_ref[...]
    cate = cate_ref[...]
    gv = gv_ref[...]

    f32 = jnp.float32
    hi = lax.Precision.HIGHEST

    cg = lax.dot_general(cate, gv, (((1,), (1,)), ((), ())),
                         preferred_element_type=f32, precision=hi)
    onehot = (cid == lax.broadcasted_iota(jnp.int32, (cid.shape[0], NUM_CATES), 1)
              ).astype(f32)
    logits = lax.dot_general(onehot, cg, (((1,), (0,)), ((), ())),
                             preferred_element_type=f32, precision=hi)
    e = jnp.exp(logits * (1.0 / TEMP))
    denom = e[:, 0:1] + e[:, 1:2]
    g0 = e[:, 0:1] / denom
    g1 = e[:, 1:2] / denom

    t = jnp.sum(item * item, axis=1, keepdims=True)
    m = jnp.concatenate(
        [item * (2.0 * g0), item * (2.0 * g1), -g0, -g1, -t], axis=1)

    s0 = jnp.sum(u[:, :DIM] * u[:, :DIM], axis=1, keepdims=True)
    s1 = jnp.sum(u[:, DIM:] * u[:, DIM:], axis=1, keepdims=True)
    ones = jnp.ones_like(s0)
    a = jnp.concatenate([u, s0, s1, ones], axis=1)

    out_ref[...] = lax.dot_general(a, m, (((1,), (1,)), ((), ())),
                                   preferred_element_type=f32, precision=hi)


def _tc_call(u128, uid_col, cid_col, item_table, cate_table, gate_vectors):
    B = u128.shape[0]
    N = item_table.shape[0]
    return pl.pallas_call(
        _tc_body,
        out_shape=jax.ShapeDtypeStruct((B, N), jnp.float32),
    )(u128, uid_col, cid_col, item_table, cate_table, gate_vectors)


@jax.jit
def kernel(user_ids, cate_ids, user_table, item_table, cate_table, gate_vectors):
    B = user_ids.shape[0]
    uid = user_ids.astype(jnp.int32)
    table2 = user_table.reshape(user_table.shape[0] // 2, 2 * user_table.shape[1])
    u128 = _sc_gather_fn(B)(table2, uid)
    uid_col = uid.reshape(-1, 1)
    cid_col = cate_ids.astype(jnp.int32).reshape(-1, 1)
    return _tc_call(u128, uid_col, cid_col, item_table, cate_table, gate_vectors)


# speedup vs baseline: 1.7075x; 1.7075x over previous
import functools

import jax
import jax.numpy as jnp
from jax import lax
from jax.experimental import pallas as pl
from jax.experimental.pallas import tpu as pltpu
from jax.experimental.pallas import tpu_sc as plsc

K = 2
DIM = 32
TEMP = 0.1
NUM_CATES = 100


def _sc_gather_fn(B, D):
    info = plsc.get_sparse_core_info()
    NC, NS = info.num_cores, info.num_subcores
    NW = NC * NS
    b_per_w = B // NW
    mesh = plsc.VectorSubcoreMesh(core_axis_name="c", subcore_axis_name="s")

    @functools.partial(
        pl.kernel,
        mesh=mesh,
        out_type=jax.ShapeDtypeStruct((B, D), jnp.float32),
        scratch_types=[
            pltpu.SMEM((b_per_w,), jnp.int32),
            pltpu.VMEM((b_per_w,), jnp.int32),
            pltpu.VMEM((b_per_w, D), jnp.float32),
            pltpu.SemaphoreType.DMA,
        ],
    )
    def gather(table_hbm, idx_hbm, out_hbm, ids_s, ids_v, rows_v, sem):
        wid = lax.axis_index("s") * NC + lax.axis_index("c")
        base = wid * b_per_w
        pltpu.sync_copy(idx_hbm.at[pl.ds(base, b_per_w)], ids_v)
        copies = []
        for j in range(b_per_w // 16):
            vec = ids_v[pl.ds(j * 16, 16)]
            for i in range(16):
                copies.append(pltpu.async_copy(
                    table_hbm.at[pl.ds(vec[i], 1)],
                    rows_v.at[pl.ds(j * 16 + i, 1)], sem))
        for c in copies:
            c.wait()
        pltpu.sync_copy(rows_v, out_hbm.at[pl.ds(base, b_per_w)])

    return gather


def _tc_body(u_ref, cid_ref, item_ref, cate_ref, gv_ref, out_ref):
    u = u_ref[...]
    cid = cid_ref[...]
    item = item_ref[...]
    cate = cate_ref[...]
    gv = gv_ref[...]

    f32 = jnp.float32
    hi = lax.Precision.HIGHEST

    cg = lax.dot_general(cate, gv, (((1,), (1,)), ((), ())),
                         preferred_element_type=f32, precision=hi)
    onehot = (cid == lax.broadcasted_iota(jnp.int32, (cid.shape[0], NUM_CATES), 1)
              ).astype(f32)
    logits = lax.dot_general(onehot, cg, (((1,), (0,)), ((), ())),
                             preferred_element_type=f32, precision=hi)
    e = jnp.exp(logits * (1.0 / TEMP))
    denom = e[:, 0:1] + e[:, 1:2]
    g0 = e[:, 0:1] / denom
    g1 = e[:, 1:2] / denom

    t = jnp.sum(item * item, axis=1, keepdims=True)
    m = jnp.concatenate(
        [item * (2.0 * g0), item * (2.0 * g1), -g0, -g1, -t], axis=1)

    s0 = jnp.sum(u[:, :DIM] * u[:, :DIM], axis=1, keepdims=True)
    s1 = jnp.sum(u[:, DIM:] * u[:, DIM:], axis=1, keepdims=True)
    ones = jnp.ones_like(s0)
    a = jnp.concatenate([u, s0, s1, ones], axis=1)

    out_ref[...] = lax.dot_general(a, m, (((1,), (1,)), ((), ())),
                                   preferred_element_type=f32, precision=hi)


def _tc_call(u, cid_col, item_table, cate_table, gate_vectors):
    B = u.shape[0]
    N = item_table.shape[0]
    return pl.pallas_call(
        _tc_body,
        out_shape=jax.ShapeDtypeStruct((B, N), jnp.float32),
    )(u, cid_col, item_table, cate_table, gate_vectors)


@jax.jit
def kernel(user_ids, cate_ids, user_table, item_table, cate_table, gate_vectors):
    B = user_ids.shape[0]
    D = user_table.shape[1]
    u = _sc_gather_fn(B, D)(user_table, user_ids.astype(jnp.int32))
    cid_col = cate_ids.astype(jnp.int32).reshape(-1, 1)
    return _tc_call(u, cid_col, item_table, cate_table, gate_vectors)
